# baseline (speedup 1.0000x reference)
import functools

import jax
import jax.numpy as jnp
from jax import lax
from jax.experimental import pallas as pl
from jax.experimental.pallas import tpu as pltpu
from jax.experimental.pallas import tpu_sc as plsc

TOKENS = 4096
INPUT_SIZE = 2048
HIDDEN_SIZE = 2048
K = 32
BT = 256

NC = 2
NS = 16
L = 16
NW = NC * NS
HALF = TOKENS // 2
RPW = HALF // NW
NG = RPW // L
NB = 256


def _pre_block(x_ref, peb_ref, wt_ref, b1_ref, o_ref):
    xc = x_ref[...] - peb_ref[...]
    o_ref[...] = (
        jnp.dot(xc, wt_ref[...], preferred_element_type=jnp.float32)
        + b1_ref[...]
    )


def _rec_block(pre_ref, thr_ref, w_ref, b2_ref, o_ref):
    pre = pre_ref[...]
    a = jnp.where(pre >= thr_ref[...], pre, jnp.float32(0.0))
    o_ref[...] = (
        jnp.dot(a, w_ref[...], preferred_element_type=jnp.float32) + b2_ref[...]
    )


def _sc_topk_body(pre_hbm, thr_hbm, buf_v, skey_v, hist_v, stage_v):
    cid = lax.axis_index("c")
    sid = lax.axis_index("s")
    wid = sid * NC + cid
    lane = lax.broadcasted_iota(jnp.int32, (L,), 0)
    ones = jnp.ones((L,), jnp.int32)
    zeros16 = jnp.zeros((L,), jnp.int32)

    lane_row = lane * jnp.int32(HIDDEN_SIZE)

    @plsc.parallel_loop(0, NB, 1, unroll=8)
    def init_zero(b):
        hist_v[pl.ds(b * L, L)] = zeros16

    def per_group(g, _):
        base = wid * RPW + g * L
        pltpu.sync_copy(
            pre_hbm.at[pl.ds(base * HIDDEN_SIZE, L * HIDDEN_SIZE)], buf_v
        )

        hi = zeros16
        krem = jnp.full((L,), K, jnp.int32)

        for p in range(4):
            shift = 24 - 8 * p

            if p == 0:
                @plsc.parallel_loop(0, HIDDEN_SIZE, 1, unroll=8)
                def hist0_body(c):
                    colv = jnp.full((L,), c, jnp.int32)
                    v = plsc.load_gather(buf_v, [lane_row + colv])
                    kb = plsc.bitcast(v, jnp.int32)
                    s = kb ^ (jnp.int32(0x7FFFFFFF) & (kb >> 31))
                    skey_v[pl.ds(c * L, L)] = s
                    bucket = ((s >> 24) & jnp.int32(255)) ^ jnp.int32(128)
                    plsc.addupdate_scatter(hist_v, [bucket * L + lane], ones)
            else:
                @plsc.parallel_loop(0, HIDDEN_SIZE, 1, unroll=8)
                def hist_body(c, _hi=hi, _shift=shift):
                    s = skey_v[pl.ds(c * L, L)]
                    bucket = (s >> _shift) & jnp.int32(255)
                    m = (s >> (_shift + 8)) == _hi
                    plsc.addupdate_scatter(
                        hist_v, [bucket * L + lane], ones, mask=m
                    )

            def scan_body(i, carry, _krem=krem):
                acc, bfound, knext = carry
                b = 255 - i
                cnt = hist_v[pl.ds(b * L, L)]
                hist_v[pl.ds(b * L, L)] = zeros16
                newacc = acc + cnt
                crossed = (acc < _krem) & (newacc >= _krem)
                bvec = jnp.full((L,), b, jnp.int32)
                bfound = jnp.where(crossed, bvec, bfound)
                knext = jnp.where(crossed, _krem - acc, knext)
                return newacc, bfound, knext

            _, bfound, knext = plsc.parallel_loop(
                0, NB, 1, unroll=8, carry=(zeros16, zeros16, zeros16)
            )(scan_body)

            if p == 0:
                r = bfound ^ jnp.int32(128)
                hi = (r << 24) >> 24
            else:
                r = bfound
                hi = (hi << 8) | r
            krem = knext

        s_thr = hi
        f_thr = plsc.bitcast(
            s_thr ^ (jnp.int32(0x7FFFFFFF) & (s_thr >> 31)), jnp.float32
        )
        stage_v[...] = f_thr
        pltpu.sync_copy(stage_v, thr_hbm.at[pl.ds(base, L)])
        return 0

    lax.fori_loop(0, NG, per_group, 0)


_sc_topk = functools.partial(
    pl.kernel,
    out_type=jax.ShapeDtypeStruct((HALF,), jnp.float32),
    mesh=plsc.VectorSubcoreMesh(core_axis_name="c", subcore_axis_name="s"),
    compiler_params=pltpu.CompilerParams(needs_layout_passes=False),
    scratch_types=[
        pltpu.VMEM((L * HIDDEN_SIZE,), jnp.float32),
        pltpu.VMEM((HIDDEN_SIZE * L,), jnp.int32),
        pltpu.VMEM((NB * L,), jnp.int32),
        pltpu.VMEM((L,), jnp.float32),
    ],
)(_sc_topk_body)


def _tc_pre(x, peb2, WT, b12):
    grid = (HALF // BT,)
    return pl.pallas_call(
        _pre_block,
        grid=grid,
        in_specs=[
            pl.BlockSpec((BT, INPUT_SIZE), lambda i: (i, 0)),
            pl.BlockSpec((1, HIDDEN_SIZE), lambda i: (0, 0)),
            pl.BlockSpec((INPUT_SIZE, HIDDEN_SIZE), lambda i: (0, 0)),
            pl.BlockSpec((1, HIDDEN_SIZE), lambda i: (0, 0)),
        ],
        out_specs=pl.BlockSpec((BT, HIDDEN_SIZE), lambda i: (i, 0)),
        out_shape=jax.ShapeDtypeStruct((HALF, HIDDEN_SIZE), jnp.float32),
        compiler_params=pltpu.CompilerParams(
            dimension_semantics=("arbitrary",),
        ),
    )(x, peb2, WT, b12)


def _tc_rec(pre, thr, W, b22):
    grid = (HALF // BT,)
    return pl.pallas_call(
        _rec_block,
        grid=grid,
        in_specs=[
            pl.BlockSpec((BT, HIDDEN_SIZE), lambda i: (i, 0)),
            pl.BlockSpec((BT, 1), lambda i: (i, 0)),
            pl.BlockSpec((HIDDEN_SIZE, INPUT_SIZE), lambda i: (0, 0)),
            pl.BlockSpec((1, INPUT_SIZE), lambda i: (0, 0)),
        ],
        out_specs=pl.BlockSpec((BT, INPUT_SIZE), lambda i: (i, 0)),
        out_shape=jax.ShapeDtypeStruct((HALF, INPUT_SIZE), jnp.float32),
        compiler_params=pltpu.CompilerParams(
            dimension_semantics=("arbitrary",),
        ),
    )(pre, thr.reshape(HALF, 1), W, b22)


@jax.jit
def _sae_hybrid(x, peb2, WT, W, b12, b22):
    xa = lax.slice_in_dim(x, 0, HALF, axis=0)
    xb = lax.slice_in_dim(x, HALF, TOKENS, axis=0)
    pre_a = _tc_pre(xa, peb2, WT, b12)
    thr_a = _sc_topk(pre_a.reshape(HALF * HIDDEN_SIZE))
    pre_b = _tc_pre(xb, peb2, WT, b12)
    out_a = _tc_rec(pre_a, thr_a, W, b22)
    thr_b = _sc_topk(pre_b.reshape(HALF * HIDDEN_SIZE))
    out_b = _tc_rec(pre_b, thr_b, W, b22)
    return jnp.concatenate([out_a, out_b], axis=0)


def kernel(x, pre_encode_b, W, WT, b1, b2):
    peb2 = pre_encode_b.reshape(1, HIDDEN_SIZE)
    b12 = b1.reshape(1, HIDDEN_SIZE)
    b22 = b2.reshape(1, INPUT_SIZE)
    return _sae_hybrid(x, peb2, WT, W, b12, b22)


# speedup vs baseline: 1.1814x; 1.1814x over previous
import functools

import jax
import jax.numpy as jnp
from jax import lax
from jax.experimental import pallas as pl
from jax.experimental.pallas import tpu as pltpu
from jax.experimental.pallas import tpu_sc as plsc

TOKENS = 4096
INPUT_SIZE = 2048
HIDDEN_SIZE = 2048
K = 32
BT = 256

NC = 2
NS = 16
L = 16
NW = NC * NS
RPW = TOKENS // NW
NG = RPW // L
NB = 256


def _pre_block(x_ref, peb_ref, wt_ref, b1_ref, o_ref):
    xc = x_ref[...] - peb_ref[...]
    o_ref[...] = (
        jnp.dot(xc, wt_ref[...], preferred_element_type=jnp.float32)
        + b1_ref[...]
    )


def _rec_block(pre_ref, thr_ref, w_ref, b2_ref, o_ref):
    pre = pre_ref[...]
    a = jnp.where(pre >= thr_ref[...], pre, jnp.float32(0.0))
    o_ref[...] = (
        jnp.dot(a, w_ref[...], preferred_element_type=jnp.float32) + b2_ref[...]
    )


def _sc_topk_body(pre_hbm, thr_hbm, buf_v, skey_v, hist_v, stage_v):
    cid = lax.axis_index("c")
    sid = lax.axis_index("s")
    wid = sid * NC + cid
    lane = lax.broadcasted_iota(jnp.int32, (L,), 0)
    ones = jnp.ones((L,), jnp.int32)
    zeros16 = jnp.zeros((L,), jnp.int32)

    lane_row = lane * jnp.int32(HIDDEN_SIZE)

    @plsc.parallel_loop(0, NB, 1, unroll=8)
    def init_zero(b):
        hist_v[pl.ds(b * L, L)] = zeros16

    def per_group(g, _):
        base = wid * RPW + g * L
        pltpu.sync_copy(
            pre_hbm.at[pl.ds(base * HIDDEN_SIZE, L * HIDDEN_SIZE)], buf_v
        )

        hi = zeros16
        krem = jnp.full((L,), K, jnp.int32)

        for p in range(4):
            shift = 24 - 8 * p

            if p == 0:
                @plsc.parallel_loop(0, HIDDEN_SIZE, 1, unroll=8)
                def hist0_body(c):
                    colv = jnp.full((L,), c, jnp.int32)
                    v = plsc.load_gather(buf_v, [lane_row + colv])
                    kb = plsc.bitcast(v, jnp.int32)
                    s = kb ^ (jnp.int32(0x7FFFFFFF) & (kb >> 31))
                    skey_v[pl.ds(c * L, L)] = s
                    bucket = ((s >> 24) & jnp.int32(255)) ^ jnp.int32(128)
                    plsc.addupdate_scatter(hist_v, [bucket * L + lane], ones)
            else:
                @plsc.parallel_loop(0, HIDDEN_SIZE, 1, unroll=8)
                def hist_body(c, _hi=hi, _shift=shift):
                    s = skey_v[pl.ds(c * L, L)]
                    bucket = (s >> _shift) & jnp.int32(255)
                    m = (s >> (_shift + 8)) == _hi
                    plsc.addupdate_scatter(
                        hist_v, [bucket * L + lane], ones, mask=m
                    )

            def scan_body(i, carry, _krem=krem):
                acc, bfound, knext = carry
                b = 255 - i
                cnt = hist_v[pl.ds(b * L, L)]
                hist_v[pl.ds(b * L, L)] = zeros16
                newacc = acc + cnt
                crossed = (acc < _krem) & (newacc >= _krem)
                bvec = jnp.full((L,), b, jnp.int32)
                bfound = jnp.where(crossed, bvec, bfound)
                knext = jnp.where(crossed, _krem - acc, knext)
                return newacc, bfound, knext

            _, bfound, knext = plsc.parallel_loop(
                0, NB, 1, unroll=8, carry=(zeros16, zeros16, zeros16)
            )(scan_body)

            if p == 0:
                r = bfound ^ jnp.int32(128)
                hi = (r << 24) >> 24
            else:
                r = bfound
                hi = (hi << 8) | r
            krem = knext

        s_thr = hi
        f_thr = plsc.bitcast(
            s_thr ^ (jnp.int32(0x7FFFFFFF) & (s_thr >> 31)), jnp.float32
        )
        stage_v[...] = f_thr
        pltpu.sync_copy(stage_v, thr_hbm.at[pl.ds(base, L)])
        return 0

    lax.fori_loop(0, NG, per_group, 0)


_sc_topk = functools.partial(
    pl.kernel,
    out_type=jax.ShapeDtypeStruct((TOKENS,), jnp.float32),
    mesh=plsc.VectorSubcoreMesh(core_axis_name="c", subcore_axis_name="s"),
    compiler_params=pltpu.CompilerParams(needs_layout_passes=False),
    scratch_types=[
        pltpu.VMEM((L * HIDDEN_SIZE,), jnp.float32),
        pltpu.VMEM((HIDDEN_SIZE * L,), jnp.int32),
        pltpu.VMEM((NB * L,), jnp.int32),
        pltpu.VMEM((L,), jnp.float32),
    ],
)(_sc_topk_body)


@jax.jit
def _sae_hybrid(x, peb2, WT, W, b12, b22):
    grid = (TOKENS // BT,)
    pre = pl.pallas_call(
        _pre_block,
        grid=grid,
        in_specs=[
            pl.BlockSpec((BT, INPUT_SIZE), lambda i: (i, 0)),
            pl.BlockSpec((1, HIDDEN_SIZE), lambda i: (0, 0)),
            pl.BlockSpec((INPUT_SIZE, HIDDEN_SIZE), lambda i: (0, 0)),
            pl.BlockSpec((1, HIDDEN_SIZE), lambda i: (0, 0)),
        ],
        out_specs=pl.BlockSpec((BT, HIDDEN_SIZE), lambda i: (i, 0)),
        out_shape=jax.ShapeDtypeStruct((TOKENS, HIDDEN_SIZE), jnp.float32),
        compiler_params=pltpu.CompilerParams(
            dimension_semantics=("arbitrary",),
        ),
    )(x, peb2, WT, b12)

    thr = _sc_topk(pre.reshape(TOKENS * HIDDEN_SIZE))

    return pl.pallas_call(
        _rec_block,
        grid=grid,
        in_specs=[
            pl.BlockSpec((BT, HIDDEN_SIZE), lambda i: (i, 0)),
            pl.BlockSpec((BT, 1), lambda i: (i, 0)),
            pl.BlockSpec((HIDDEN_SIZE, INPUT_SIZE), lambda i: (0, 0)),
            pl.BlockSpec((1, INPUT_SIZE), lambda i: (0, 0)),
        ],
        out_specs=pl.BlockSpec((BT, INPUT_SIZE), lambda i: (i, 0)),
        out_shape=jax.ShapeDtypeStruct((TOKENS, INPUT_SIZE), jnp.float32),
        compiler_params=pltpu.CompilerParams(
            dimension_semantics=("arbitrary",),
        ),
    )(pre, thr.reshape(TOKENS, 1), W, b22)


def kernel(x, pre_encode_b, W, WT, b1, b2):
    peb2 = pre_encode_b.reshape(1, HIDDEN_SIZE)
    b12 = b1.reshape(1, HIDDEN_SIZE)
    b22 = b2.reshape(1, INPUT_SIZE)
    return _sae_hybrid(x, peb2, WT, W, b12, b22)


# speedup vs baseline: 1.2293x; 1.0405x over previous
import functools

import jax
import jax.numpy as jnp
from jax import lax
from jax.experimental import pallas as pl
from jax.experimental.pallas import tpu as pltpu
from jax.experimental.pallas import tpu_sc as plsc

TOKENS = 4096
INPUT_SIZE = 2048
HIDDEN_SIZE = 2048
K = 32
BT = 256

NC = 2
NS = 16
L = 16
NW = NC * NS
RPW = TOKENS // NW
NG = RPW // L
NB = 256


def _pre_block(x_ref, peb_ref, wt_ref, b1_ref, o_ref):
    xc = x_ref[...] - peb_ref[...]
    o_ref[...] = (
        jnp.dot(xc, wt_ref[...], preferred_element_type=jnp.float32)
        + b1_ref[...]
    )


def _rec_block(pre_ref, thr_ref, w_ref, b2_ref, o_ref):
    pre = pre_ref[...]
    a = jnp.where(pre >= thr_ref[...], pre, jnp.float32(0.0))
    o_ref[...] = (
        jnp.dot(a, w_ref[...], preferred_element_type=jnp.float32) + b2_ref[...]
    )


def _sc_topk_body(pre_hbm, thr_hbm, buf_v, buf2_v, skey_v, hist_v, stage_v, sem0, sem1):
    cid = lax.axis_index("c")
    sid = lax.axis_index("s")
    wid = sid * NC + cid
    lane = lax.broadcasted_iota(jnp.int32, (L,), 0)
    ones = jnp.ones((L,), jnp.int32)
    zeros16 = jnp.zeros((L,), jnp.int32)

    lane_row = lane * jnp.int32(HIDDEN_SIZE)

    @plsc.parallel_loop(0, NB, 1, unroll=8)
    def init_zero(b):
        hist_v[pl.ds(b * L, L)] = zeros16

    def _src(g):
        base = wid * RPW + g * L
        return pre_hbm.at[pl.ds(base * HIDDEN_SIZE, L * HIDDEN_SIZE)]

    def process_group(g, buf):
        base = wid * RPW + g * L

        hi = zeros16
        krem = jnp.full((L,), K, jnp.int32)

        for p in range(4):
            shift = 24 - 8 * p

            if p == 0:
                @plsc.parallel_loop(0, HIDDEN_SIZE, 1, unroll=8)
                def hist0_body(c):
                    colv = jnp.full((L,), c, jnp.int32)
                    v = plsc.load_gather(buf, [lane_row + colv])
                    kb = plsc.bitcast(v, jnp.int32)
                    s = kb ^ (jnp.int32(0x7FFFFFFF) & (kb >> 31))
                    skey_v[pl.ds(c * L, L)] = s
                    bucket = ((s >> 24) & jnp.int32(255)) ^ jnp.int32(128)
                    plsc.addupdate_scatter(hist_v, [bucket * L + lane], ones)
            else:
                @plsc.parallel_loop(0, HIDDEN_SIZE, 1, unroll=8)
                def hist_body(c, _hi=hi, _shift=shift):
                    s = skey_v[pl.ds(c * L, L)]
                    bucket = (s >> _shift) & jnp.int32(255)
                    m = (s >> (_shift + 8)) == _hi
                    plsc.addupdate_scatter(
                        hist_v, [bucket * L + lane], ones, mask=m
                    )

            def scan_body(i, carry, _krem=krem):
                acc, bfound, knext = carry
                b = 255 - i
                cnt = hist_v[pl.ds(b * L, L)]
                hist_v[pl.ds(b * L, L)] = zeros16
                newacc = acc + cnt
                crossed = (acc < _krem) & (newacc >= _krem)
                bvec = jnp.full((L,), b, jnp.int32)
                bfound = jnp.where(crossed, bvec, bfound)
                knext = jnp.where(crossed, _krem - acc, knext)
                return newacc, bfound, knext

            _, bfound, knext = plsc.parallel_loop(
                0, NB, 1, unroll=8, carry=(zeros16, zeros16, zeros16)
            )(scan_body)

            if p == 0:
                r = bfound ^ jnp.int32(128)
                hi = (r << 24) >> 24
            else:
                r = bfound
                hi = (hi << 8) | r
            krem = knext

        s_thr = hi
        f_thr = plsc.bitcast(
            s_thr ^ (jnp.int32(0x7FFFFFFF) & (s_thr >> 31)), jnp.float32
        )
        stage_v[...] = f_thr
        pltpu.sync_copy(stage_v, thr_hbm.at[pl.ds(base, L)])

    pltpu.async_copy(_src(0), buf_v, sem0)

    def pair_body(j, _):
        g0 = 2 * j
        pltpu.make_async_copy(_src(g0), buf_v, sem0).wait()
        pltpu.async_copy(_src(g0 + 1), buf2_v, sem1)
        process_group(g0, buf_v)

        pltpu.make_async_copy(_src(g0 + 1), buf2_v, sem1).wait()

        @pl.when(g0 + 2 < NG)
        def _():
            pltpu.async_copy(_src(g0 + 2), buf_v, sem0)

        process_group(g0 + 1, buf2_v)
        return 0

    lax.fori_loop(0, NG // 2, pair_body, 0)


_sc_topk = functools.partial(
    pl.kernel,
    out_type=jax.ShapeDtypeStruct((TOKENS,), jnp.float32),
    mesh=plsc.VectorSubcoreMesh(core_axis_name="c", subcore_axis_name="s"),
    compiler_params=pltpu.CompilerParams(needs_layout_passes=False),
    scratch_types=[
        pltpu.VMEM((L * HIDDEN_SIZE,), jnp.float32),
        pltpu.VMEM((L * HIDDEN_SIZE,), jnp.float32),
        pltpu.VMEM((HIDDEN_SIZE * L,), jnp.int32),
        pltpu.VMEM((NB * L,), jnp.int32),
        pltpu.VMEM((L,), jnp.float32),
        pltpu.SemaphoreType.DMA,
        pltpu.SemaphoreType.DMA,
    ],
)(_sc_topk_body)


@jax.jit
def _sae_hybrid(x, peb2, WT, W, b12, b22):
    grid = (TOKENS // BT,)
    pre = pl.pallas_call(
        _pre_block,
        grid=grid,
        in_specs=[
            pl.BlockSpec((BT, INPUT_SIZE), lambda i: (i, 0)),
            pl.BlockSpec((1, HIDDEN_SIZE), lambda i: (0, 0)),
            pl.BlockSpec((INPUT_SIZE, HIDDEN_SIZE), lambda i: (0, 0)),
            pl.BlockSpec((1, HIDDEN_SIZE), lambda i: (0, 0)),
        ],
        out_specs=pl.BlockSpec((BT, HIDDEN_SIZE), lambda i: (i, 0)),
        out_shape=jax.ShapeDtypeStruct((TOKENS, HIDDEN_SIZE), jnp.float32),
        compiler_params=pltpu.CompilerParams(
            dimension_semantics=("arbitrary",),
        ),
    )(x, peb2, WT, b12)

    thr = _sc_topk(pre.reshape(TOKENS * HIDDEN_SIZE))

    return pl.pallas_call(
        _rec_block,
        grid=grid,
        in_specs=[
            pl.BlockSpec((BT, HIDDEN_SIZE), lambda i: (i, 0)),
            pl.BlockSpec((BT, 1), lambda i: (i, 0)),
            pl.BlockSpec((HIDDEN_SIZE, INPUT_SIZE), lambda i: (0, 0)),
            pl.BlockSpec((1, INPUT_SIZE), lambda i: (0, 0)),
        ],
        out_specs=pl.BlockSpec((BT, INPUT_SIZE), lambda i: (i, 0)),
        out_shape=jax.ShapeDtypeStruct((TOKENS, INPUT_SIZE), jnp.float32),
        compiler_params=pltpu.CompilerParams(
            dimension_semantics=("arbitrary",),
        ),
    )(pre, thr.reshape(TOKENS, 1), W, b22)


def kernel(x, pre_encode_b, W, WT, b1, b2):
    peb2 = pre_encode_b.reshape(1, HIDDEN_SIZE)
    b12 = b1.reshape(1, HIDDEN_SIZE)
    b22 = b2.reshape(1, INPUT_SIZE)
    return _sae_hybrid(x, peb2, WT, W, b12, b22)
